# Initial kernel scaffold; baseline (speedup 1.0000x reference)
import jax
import jax.numpy as jnp
from jax.experimental import pallas as pl


def kernel(x, adj, W1, b1, W2, b2, W3, b3, W4, b4):
    raise NotImplementedError("write your pallas kernel here")


# speedup vs baseline: 1.4227x; 1.4227x over previous
import jax
import jax.numpy as jnp
from jax.experimental import pallas as pl
from jax.experimental.pallas import tpu as pltpu

_ROWS_PER_BLOCK = 256


def _gcn_allpass_kernel(m0_ref, a_ref, rho_ref, out_ref, a16, ma, mb):
    nblk = pl.num_programs(0)
    i = pl.program_id(0)
    rb = a_ref.shape[0]
    n = a_ref.shape[1]
    dg = out_ref.shape[0]
    nb = dg // (m0_ref.shape[1] - dg)

    ab = a_ref[...].astype(jnp.bfloat16)
    a16[pl.ds(i * rb, rb), :] = ab
    m = (m0_ref[...] + rho_ref[0:1, :]).astype(jnp.bfloat16)
    ma[pl.ds(i * rb, rb), :] = jnp.dot(ab, m, preferred_element_type=jnp.float32)

    @pl.when(i == nblk - 1)
    def _tail():
        def one_pass(src, dst, p):
            mp = (src[...] + rho_ref[p:p + 1, :]).astype(jnp.bfloat16)
            for j in range(nblk):
                dst[j * rb:(j + 1) * rb, :] = jnp.dot(
                    a16[j * rb:(j + 1) * rb, :], mp,
                    preferred_element_type=jnp.float32)

        one_pass(ma, mb, 1)
        one_pass(mb, ma, 2)

        mp = (ma[...] + rho_ref[3:4, :]).astype(jnp.bfloat16)
        for j in range(nblk):
            res = jnp.dot(a16[j * rb:(j + 1) * rb, :], mp,
                          preferred_element_type=jnp.float32)
            comb = res[:, :dg] + jnp.concatenate([res[:, dg:]] * nb, axis=1)
            out_ref[:, j * rb:(j + 1) * rb] = comb.T


def kernel(x, adj, W1, b1, W2, b2, W3, b3, W4, b4):
    nb, in_dim, n = x.shape
    out_dim = W4.shape[0]
    f32 = jnp.float32

    c4 = W4.T
    c3 = W3.T @ c4
    c2 = W2.T @ c3
    c1 = W1.T @ c2

    rho = jnp.zeros((8, nb * out_dim + out_dim), f32)
    rho = rho.at[0, nb * out_dim:].set(b1 @ c2)
    rho = rho.at[1, nb * out_dim:].set(b2 @ c3)
    rho = rho.at[2, nb * out_dim:].set(b3 @ c4)
    rho = rho.at[3, nb * out_dim:].set(b4)

    h0 = jnp.transpose(x, (2, 0, 1))
    g0 = jnp.einsum('nbc,cd->nbd', h0, c1).reshape(n, nb * out_dim)
    m0 = jnp.concatenate([g0, jnp.zeros((n, out_dim), f32)], axis=1)

    rb = _ROWS_PER_BLOCK
    nblk = n // rb
    w = nb * out_dim + out_dim

    out = pl.pallas_call(
        _gcn_allpass_kernel,
        grid=(nblk,),
        in_specs=[
            pl.BlockSpec((n, w), lambda i: (0, 0)),
            pl.BlockSpec((rb, n), lambda i: (i, 0)),
            pl.BlockSpec((8, w), lambda i: (0, 0)),
        ],
        out_specs=pl.BlockSpec((nb * out_dim, n), lambda i: (0, 0)),
        out_shape=jax.ShapeDtypeStruct((nb * out_dim, n), f32),
        scratch_shapes=[
            pltpu.VMEM((n, n), jnp.bfloat16),
            pltpu.VMEM((n, w), f32),
            pltpu.VMEM((n, w), f32),
        ],
        compiler_params=pltpu.CompilerParams(
            vmem_limit_bytes=100 * 1024 * 1024,
        ),
    )(m0, adj, rho)

    return out.reshape(nb, out_dim, n)


# speedup vs baseline: 1.5024x; 1.0560x over previous
import jax
import jax.numpy as jnp
from jax.experimental import pallas as pl
from jax.experimental.pallas import tpu as pltpu

_ROWS_PER_BLOCK = 512


def _gcn_allpass_kernel(m0_ref, a_ref, rho_ref, out_ref, a16, ma, mb):
    nblk = pl.num_programs(0)
    i = pl.program_id(0)
    rb = a_ref.shape[0]
    n = a_ref.shape[1]
    dg = out_ref.shape[0]
    nb = dg // (m0_ref.shape[1] - dg)

    ab = a_ref[...].astype(jnp.bfloat16)
    a16[pl.ds(i * rb, rb), :] = ab
    m = (m0_ref[...] + rho_ref[0:1, :]).astype(jnp.bfloat16)
    ma[pl.ds(i * rb, rb), :] = jnp.dot(ab, m, preferred_element_type=jnp.float32)

    @pl.when(i == nblk - 1)
    def _tail():
        def one_pass(src, dst, p):
            mp = (src[...] + rho_ref[p:p + 1, :]).astype(jnp.bfloat16)
            for j in range(nblk):
                dst[j * rb:(j + 1) * rb, :] = jnp.dot(
                    a16[j * rb:(j + 1) * rb, :], mp,
                    preferred_element_type=jnp.float32)

        one_pass(ma, mb, 1)
        one_pass(mb, ma, 2)

        mp = (ma[...] + rho_ref[3:4, :]).astype(jnp.bfloat16)
        for j in range(nblk):
            res = jnp.dot(a16[j * rb:(j + 1) * rb, :], mp,
                          preferred_element_type=jnp.float32)
            comb = res[:, :dg] + jnp.concatenate([res[:, dg:]] * nb, axis=1)
            out_ref[:, j * rb:(j + 1) * rb] = comb.T


def kernel(x, adj, W1, b1, W2, b2, W3, b3, W4, b4):
    nb, in_dim, n = x.shape
    out_dim = W4.shape[0]
    f32 = jnp.float32

    c4 = W4.T
    c3 = W3.T @ c4
    c2 = W2.T @ c3
    c1 = W1.T @ c2

    rho = jnp.zeros((8, nb * out_dim + out_dim), f32)
    rho = rho.at[0, nb * out_dim:].set(b1 @ c2)
    rho = rho.at[1, nb * out_dim:].set(b2 @ c3)
    rho = rho.at[2, nb * out_dim:].set(b3 @ c4)
    rho = rho.at[3, nb * out_dim:].set(b4)

    h0 = jnp.transpose(x, (2, 0, 1))
    g0 = jnp.einsum('nbc,cd->nbd', h0, c1).reshape(n, nb * out_dim)
    m0 = jnp.concatenate([g0, jnp.zeros((n, out_dim), f32)], axis=1)

    rb = _ROWS_PER_BLOCK
    nblk = n // rb
    w = nb * out_dim + out_dim

    out = pl.pallas_call(
        _gcn_allpass_kernel,
        grid=(nblk,),
        in_specs=[
            pl.BlockSpec((n, w), lambda i: (0, 0)),
            pl.BlockSpec((rb, n), lambda i: (i, 0)),
            pl.BlockSpec((8, w), lambda i: (0, 0)),
        ],
        out_specs=pl.BlockSpec((nb * out_dim, n), lambda i: (0, 0)),
        out_shape=jax.ShapeDtypeStruct((nb * out_dim, n), f32),
        scratch_shapes=[
            pltpu.VMEM((n, n), jnp.bfloat16),
            pltpu.VMEM((n, w), f32),
            pltpu.VMEM((n, w), f32),
        ],
        compiler_params=pltpu.CompilerParams(
            vmem_limit_bytes=100 * 1024 * 1024,
        ),
    )(m0, adj, rho)

    return out.reshape(nb, out_dim, n)
